# baseline (speedup 1.0000x reference)
import functools
import jax
import jax.numpy as jnp
from jax.experimental import pallas as pl
from jax.experimental.pallas import tpu as pltpu

N = 10000
C_IN = 32
C_OUT = 32
HID = 64
K = 64
Q_TILE = 40
GRID = N // Q_TILE


def _ln(x, g, b, eps=1e-5):
    m = jnp.mean(x, axis=-1, keepdims=True)
    v = jnp.mean((x - m) * (x - m), axis=-1, keepdims=True)
    return (x - m) * jax.lax.rsqrt(v + eps) * g + b


def _gelu(x):
    return x * 0.5 * (1.0 + jax.lax.erf(x * 0.7071067811865476))


def _edge_body(feats_ref, verts_ref, nbrF_ref, nbrV_ref,
               w1a_ref, w1b_ref, w1c_ref, b1_ref, g1_ref, bt1_ref,
               w2_ref, b2_ref,
               wsa_ref, wsb_ref, wsc_ref, bs_ref, g2_ref, bt2_ref,
               ow1_ref, ob1_ref, og1_ref, obt1_ref,
               ow2_ref, ob2_ref, og2_ref, obt2_ref,
               out_ref):
    E = Q_TILE * K
    self_f = feats_ref[...]
    qv = verts_ref[...]
    nbrF = nbrF_ref[...]
    nbrV = nbrV_ref[...]

    selfe = jnp.broadcast_to(self_f[:, None, :], (Q_TILE, K, C_IN)).reshape(E, C_IN)
    rel = (nbrV.reshape(Q_TILE, K, 3) - qv[:, None, :]).reshape(E, 3)

    h = (jnp.dot(selfe, w1a_ref[...], preferred_element_type=jnp.float32)
         + jnp.dot(nbrF, w1b_ref[...], preferred_element_type=jnp.float32)
         + jnp.dot(rel, w1c_ref[...], preferred_element_type=jnp.float32)
         + b1_ref[...])
    h = _gelu(_ln(h, g1_ref[...], bt1_ref[...]))
    h2 = jnp.dot(h, w2_ref[...], preferred_element_type=jnp.float32) + b2_ref[...]
    sc = (jnp.dot(selfe, wsa_ref[...], preferred_element_type=jnp.float32)
          + jnp.dot(nbrF, wsb_ref[...], preferred_element_type=jnp.float32)
          + jnp.dot(rel, wsc_ref[...], preferred_element_type=jnp.float32)
          + bs_ref[...])
    e = _gelu(_ln(h2 + sc, g2_ref[...], bt2_ref[...]))

    red = jnp.mean(e.reshape(Q_TILE, K, C_OUT), axis=1)

    oh = jnp.dot(red, ow1_ref[...], preferred_element_type=jnp.float32) + ob1_ref[...]
    oh = _gelu(_ln(oh, og1_ref[...], obt1_ref[...]))
    oh2 = jnp.dot(oh, ow2_ref[...], preferred_element_type=jnp.float32) + ob2_ref[...]
    out_ref[...] = _gelu(_ln(oh2 + red, og2_ref[...], obt2_ref[...]))


def _rep(shape):
    return pl.BlockSpec(shape, lambda i: (0,) * len(shape))


def _edge_pallas(feats, verts, nbrF, nbrV, weights):
    E = Q_TILE * K
    in_specs = [
        pl.BlockSpec((Q_TILE, C_IN), lambda i: (i, 0)),
        pl.BlockSpec((Q_TILE, 3), lambda i: (i, 0)),
        pl.BlockSpec((E, C_IN), lambda i: (i, 0)),
        pl.BlockSpec((E, 3), lambda i: (i, 0)),
    ] + [_rep(w.shape) for w in weights]
    return pl.pallas_call(
        _edge_body,
        grid=(GRID,),
        in_specs=in_specs,
        out_specs=pl.BlockSpec((Q_TILE, C_OUT), lambda i: (i, 0)),
        out_shape=jax.ShapeDtypeStruct((N, C_OUT), jnp.float32),
    )(feats, verts, nbrF, nbrV, *weights)


def _knn_idx(in_v, k, chunk=2500, nb=16):
    vn = jnp.sum(in_v * in_v, axis=1)
    sub = N // nb
    boff = (jnp.arange(nb, dtype=jnp.int32) * sub)[None, :, None]
    qs = in_v.reshape(N // chunk, chunk, 3)

    def body(q):
        s = 2.0 * jnp.dot(q, in_v.T, precision=jax.lax.Precision.HIGHEST) - vn[None, :]
        sv, si = jax.lax.top_k(s.reshape(chunk, nb, sub), k)
        gi = (si.astype(jnp.int32) + boff).reshape(chunk, nb * k)
        _, sel = jax.lax.top_k(sv.reshape(chunk, nb * k), k)
        return jnp.take_along_axis(gi, sel, axis=1)

    idx = jax.lax.map(body, qs)
    return idx.reshape(N * k)


def kernel(vertices, features, e_w1, e_b1, e_g1, e_bt1, e_w2, e_b2, e_ws,
           e_bs, e_g2, e_bt2, o_w1, o_b1, o_g1, o_bt1, o_w2, o_b2, o_g2,
           o_bt2):
    B = vertices.shape[0]
    in_v = vertices.reshape(N, 3)
    feats = features.reshape(N, C_IN)

    idx = _knn_idx(in_v, K)
    nbrF = feats[idx]
    nbrV = in_v[idx]

    weights = (
        e_w1[:C_IN], e_w1[C_IN:2 * C_IN], e_w1[2 * C_IN:], e_b1, e_g1, e_bt1,
        e_w2, e_b2,
        e_ws[:C_IN], e_ws[C_IN:2 * C_IN], e_ws[2 * C_IN:], e_bs, e_g2, e_bt2,
        o_w1, o_b1, o_g1, o_bt1, o_w2, o_b2, o_g2, o_bt2,
    )
    out = _edge_pallas(feats, in_v, nbrF, nbrV, weights)
    return out.reshape(B, N, C_OUT)


# speedup vs baseline: 3.4933x; 3.4933x over previous
import functools
import jax
import jax.numpy as jnp
from jax.experimental import pallas as pl
from jax.experimental.pallas import tpu as pltpu

N = 10000
C_IN = 32
C_OUT = 32
HID = 64
K = 64
Q_TILE = 40
GRID = N // Q_TILE


def _ln(x, g, b, eps=1e-5):
    m = jnp.mean(x, axis=-1, keepdims=True)
    v = jnp.mean((x - m) * (x - m), axis=-1, keepdims=True)
    return (x - m) * jax.lax.rsqrt(v + eps) * g + b


def _gelu(x):
    return x * 0.5 * (1.0 + jax.lax.erf(x * 0.7071067811865476))


def _edge_body(feats_ref, verts_ref, nbrF_ref, nbrV_ref,
               w1a_ref, w1b_ref, w1c_ref, b1_ref, g1_ref, bt1_ref,
               w2_ref, b2_ref,
               wsa_ref, wsb_ref, wsc_ref, bs_ref, g2_ref, bt2_ref,
               ow1_ref, ob1_ref, og1_ref, obt1_ref,
               ow2_ref, ob2_ref, og2_ref, obt2_ref,
               out_ref):
    E = Q_TILE * K
    self_f = feats_ref[...]
    qv = verts_ref[...]
    nbrF = nbrF_ref[...]
    nbrV = nbrV_ref[...]

    selfe = jnp.broadcast_to(self_f[:, None, :], (Q_TILE, K, C_IN)).reshape(E, C_IN)
    rel = (nbrV.reshape(Q_TILE, K, 3) - qv[:, None, :]).reshape(E, 3)

    h = (jnp.dot(selfe, w1a_ref[...], preferred_element_type=jnp.float32)
         + jnp.dot(nbrF, w1b_ref[...], preferred_element_type=jnp.float32)
         + jnp.dot(rel, w1c_ref[...], preferred_element_type=jnp.float32)
         + b1_ref[...])
    h = _gelu(_ln(h, g1_ref[...], bt1_ref[...]))
    h2 = jnp.dot(h, w2_ref[...], preferred_element_type=jnp.float32) + b2_ref[...]
    sc = (jnp.dot(selfe, wsa_ref[...], preferred_element_type=jnp.float32)
          + jnp.dot(nbrF, wsb_ref[...], preferred_element_type=jnp.float32)
          + jnp.dot(rel, wsc_ref[...], preferred_element_type=jnp.float32)
          + bs_ref[...])
    e = _gelu(_ln(h2 + sc, g2_ref[...], bt2_ref[...]))

    red = jnp.mean(e.reshape(Q_TILE, K, C_OUT), axis=1)

    oh = jnp.dot(red, ow1_ref[...], preferred_element_type=jnp.float32) + ob1_ref[...]
    oh = _gelu(_ln(oh, og1_ref[...], obt1_ref[...]))
    oh2 = jnp.dot(oh, ow2_ref[...], preferred_element_type=jnp.float32) + ob2_ref[...]
    out_ref[...] = _gelu(_ln(oh2 + red, og2_ref[...], obt2_ref[...]))


def _rep(shape):
    return pl.BlockSpec(shape, lambda i: (0,) * len(shape))


def _edge_pallas(feats, verts, nbrF, nbrV, weights):
    E = Q_TILE * K
    in_specs = [
        pl.BlockSpec((Q_TILE, C_IN), lambda i: (i, 0)),
        pl.BlockSpec((Q_TILE, 3), lambda i: (i, 0)),
        pl.BlockSpec((E, C_IN), lambda i: (i, 0)),
        pl.BlockSpec((E, 3), lambda i: (i, 0)),
    ] + [_rep(w.shape) for w in weights]
    return pl.pallas_call(
        _edge_body,
        grid=(GRID,),
        in_specs=in_specs,
        out_specs=pl.BlockSpec((Q_TILE, C_OUT), lambda i: (i, 0)),
        out_shape=jax.ShapeDtypeStruct((N, C_OUT), jnp.float32),
    )(feats, verts, nbrF, nbrV, *weights)


def _knn_idx(in_v, k, chunk=2500, grp=1024, sub=10):
    vn = jnp.sum(in_v * in_v, axis=1)
    npad = grp * sub
    qs = in_v.reshape(N // chunk, chunk, 3)

    def body(q):
        s = 2.0 * jnp.dot(q, in_v.T, precision=jax.lax.Precision.HIGHEST) - vn[None, :]
        sp = jnp.pad(s, ((0, 0), (0, npad - N)), constant_values=-jnp.inf)
        sg = sp.reshape(chunk, grp, sub)
        _, gidx = jax.lax.top_k(jnp.max(sg, axis=-1), k)
        cand = jnp.take_along_axis(sg, gidx[..., None], axis=1)
        _, sel = jax.lax.top_k(cand.reshape(chunk, k * sub), k)
        g = jnp.take_along_axis(gidx, sel // sub, axis=1)
        return g * sub + (sel % sub)

    idx = jax.lax.map(body, qs)
    return idx.reshape(N * k)


def kernel(vertices, features, e_w1, e_b1, e_g1, e_bt1, e_w2, e_b2, e_ws,
           e_bs, e_g2, e_bt2, o_w1, o_b1, o_g1, o_bt1, o_w2, o_b2, o_g2,
           o_bt2):
    B = vertices.shape[0]
    in_v = vertices.reshape(N, 3)
    feats = features.reshape(N, C_IN)

    idx = _knn_idx(in_v, K)
    nbrF = feats[idx]
    nbrV = in_v[idx]

    weights = (
        e_w1[:C_IN], e_w1[C_IN:2 * C_IN], e_w1[2 * C_IN:], e_b1, e_g1, e_bt1,
        e_w2, e_b2,
        e_ws[:C_IN], e_ws[C_IN:2 * C_IN], e_ws[2 * C_IN:], e_bs, e_g2, e_bt2,
        o_w1, o_b1, o_g1, o_bt1, o_w2, o_b2, o_g2, o_bt2,
    )
    out = _edge_pallas(feats, in_v, nbrF, nbrV, weights)
    return out.reshape(B, N, C_OUT)


# speedup vs baseline: 4.8031x; 1.3750x over previous
import functools
import jax
import jax.numpy as jnp
from jax.experimental import pallas as pl
from jax.experimental.pallas import tpu as pltpu
from jax.experimental.pallas import tpu_sc as plsc

N = 10000
C_IN = 32
C_OUT = 32
HID = 64
K = 64
Q_TILE = 40
GRID = N // Q_TILE

SC_NC = 2
SC_NS = 16
SC_NW = SC_NC * SC_NS
GCH = 128
PACK = 128


def _ln(x, g, b, eps=1e-5):
    m = jnp.mean(x, axis=-1, keepdims=True)
    v = jnp.mean((x - m) * (x - m), axis=-1, keepdims=True)
    return (x - m) * jax.lax.rsqrt(v + eps) * g + b


def _gelu(x):
    return x * 0.5 * (1.0 + jax.lax.erf(x * 0.7071067811865476))


def _edge_body(feats_ref, verts_ref, nbr_ref,
               w1a_ref, w1b_ref, w1c_ref, b1_ref, g1_ref, bt1_ref,
               w2_ref, b2_ref,
               wsa_ref, wsb_ref, wsc_ref, bs_ref, g2_ref, bt2_ref,
               ow1_ref, ob1_ref, og1_ref, obt1_ref,
               ow2_ref, ob2_ref, og2_ref, obt2_ref,
               out_ref):
    E = Q_TILE * K
    self_f = feats_ref[...]
    qv = verts_ref[...]
    nbr = nbr_ref[...]
    nbrF = nbr[:, :C_IN]
    nbrV = nbr[:, C_IN:C_IN + 3]

    selfe = jnp.broadcast_to(self_f[:, None, :], (Q_TILE, K, C_IN)).reshape(E, C_IN)
    rel = (nbrV.reshape(Q_TILE, K, 3) - qv[:, None, :]).reshape(E, 3)

    h = (jnp.dot(selfe, w1a_ref[...], preferred_element_type=jnp.float32)
         + jnp.dot(nbrF, w1b_ref[...], preferred_element_type=jnp.float32)
         + jnp.dot(rel, w1c_ref[...], preferred_element_type=jnp.float32)
         + b1_ref[...])
    h = _gelu(_ln(h, g1_ref[...], bt1_ref[...]))
    h2 = jnp.dot(h, w2_ref[...], preferred_element_type=jnp.float32) + b2_ref[...]
    sc = (jnp.dot(selfe, wsa_ref[...], preferred_element_type=jnp.float32)
          + jnp.dot(nbrF, wsb_ref[...], preferred_element_type=jnp.float32)
          + jnp.dot(rel, wsc_ref[...], preferred_element_type=jnp.float32)
          + bs_ref[...])
    e = _gelu(_ln(h2 + sc, g2_ref[...], bt2_ref[...]))

    red = jnp.mean(e.reshape(Q_TILE, K, C_OUT), axis=1)

    oh = jnp.dot(red, ow1_ref[...], preferred_element_type=jnp.float32) + ob1_ref[...]
    oh = _gelu(_ln(oh, og1_ref[...], obt1_ref[...]))
    oh2 = jnp.dot(oh, ow2_ref[...], preferred_element_type=jnp.float32) + ob2_ref[...]
    out_ref[...] = _gelu(_ln(oh2 + red, og2_ref[...], obt2_ref[...]))


def _rep(shape):
    return pl.BlockSpec(shape, lambda i: (0,) * len(shape))


def _edge_pallas(feats, verts, nbr_packed, weights):
    E = Q_TILE * K
    in_specs = [
        pl.BlockSpec((Q_TILE, C_IN), lambda i: (i, 0)),
        pl.BlockSpec((Q_TILE, 3), lambda i: (i, 0)),
        pl.BlockSpec((E, PACK), lambda i: (i, 0)),
    ] + [_rep(w.shape) for w in weights]
    return pl.pallas_call(
        _edge_body,
        grid=(GRID,),
        in_specs=in_specs,
        out_specs=pl.BlockSpec((Q_TILE, C_OUT), lambda i: (i, 0)),
        out_shape=jax.ShapeDtypeStruct((N, C_OUT), jnp.float32),
    )(feats, verts, nbr_packed, *weights)


def _sc_gather(table, idx):
    B = idx.shape[0]
    nch = B // GCH
    base = nch // SC_NW
    extra = nch - base * SC_NW
    mesh = plsc.VectorSubcoreMesh(core_axis_name="c", subcore_axis_name="s")

    @functools.partial(
        pl.kernel, mesh=mesh,
        out_type=jax.ShapeDtypeStruct((B, PACK), jnp.float32),
        scratch_types=[
            pltpu.VMEM((GCH,), jnp.int32),
            pltpu.VMEM((GCH, PACK), jnp.float32),
            pltpu.SemaphoreType.DMA,
        ],
    )
    def k(table_hbm, idx_hbm, out_hbm, idx_v, rows_v, sem):
        wid = jax.lax.axis_index("s") * SC_NC + jax.lax.axis_index("c")
        trips = base + jnp.where(wid < extra, 1, 0)

        def body(i, carry):
            off = (wid + i * SC_NW) * GCH
            pltpu.sync_copy(idx_hbm.at[pl.ds(off, GCH)], idx_v)
            pltpu.async_copy(table_hbm.at[idx_v], rows_v, sem).wait()
            pltpu.sync_copy(rows_v, out_hbm.at[pl.ds(off, GCH)])
            return carry

        jax.lax.fori_loop(0, trips, body, 0)

    return k(table, idx)


def _knn_idx(in_v, k, chunk=2500, grp=1024, sub=10):
    vn = jnp.sum(in_v * in_v, axis=1)
    npad = grp * sub
    qs = in_v.reshape(N // chunk, chunk, 3)

    def body(q):
        s = 2.0 * jnp.dot(q, in_v.T, precision=jax.lax.Precision.HIGHEST) - vn[None, :]
        sp = jnp.pad(s, ((0, 0), (0, npad - N)), constant_values=-jnp.inf)
        sg = sp.reshape(chunk, grp, sub)
        _, gidx = jax.lax.top_k(jnp.max(sg, axis=-1), k)
        cand = jnp.take_along_axis(sg, gidx[..., None], axis=1)
        _, sel = jax.lax.top_k(cand.reshape(chunk, k * sub), k)
        g = jnp.take_along_axis(gidx, sel // sub, axis=1)
        return g * sub + (sel % sub)

    idx = jax.lax.map(body, qs)
    return idx.reshape(N * k)


def kernel(vertices, features, e_w1, e_b1, e_g1, e_bt1, e_w2, e_b2, e_ws,
           e_bs, e_g2, e_bt2, o_w1, o_b1, o_g1, o_bt1, o_w2, o_b2, o_g2,
           o_bt2):
    B = vertices.shape[0]
    in_v = vertices.reshape(N, 3)
    feats = features.reshape(N, C_IN)

    idx = _knn_idx(in_v, K)
    table = jnp.concatenate(
        [feats, in_v, jnp.zeros((N, PACK - C_IN - 3), jnp.float32)], axis=1)
    nbr_packed = _sc_gather(table, idx)

    weights = (
        e_w1[:C_IN], e_w1[C_IN:2 * C_IN], e_w1[2 * C_IN:], e_b1, e_g1, e_bt1,
        e_w2, e_b2,
        e_ws[:C_IN], e_ws[C_IN:2 * C_IN], e_ws[2 * C_IN:], e_bs, e_g2, e_bt2,
        o_w1, o_b1, o_g1, o_bt1, o_w2, o_b2, o_g2, o_bt2,
    )
    out = _edge_pallas(feats, in_v, nbr_packed, weights)
    return out.reshape(B, N, C_OUT)


# speedup vs baseline: 5.4932x; 1.1437x over previous
import functools
import jax
import jax.numpy as jnp
from jax.experimental import pallas as pl
from jax.experimental.pallas import tpu as pltpu
from jax.experimental.pallas import tpu_sc as plsc

N = 10000
C_IN = 32
C_OUT = 32
HID = 64
K = 64
Q_TILE = 40
GRID = N // Q_TILE

SC_NC = 2
SC_NS = 16
SC_NW = SC_NC * SC_NS
GCH = 128
PACK = 128


def _ln(x, g, b, eps=1e-5):
    m = jnp.mean(x, axis=-1, keepdims=True)
    v = jnp.mean((x - m) * (x - m), axis=-1, keepdims=True)
    return (x - m) * jax.lax.rsqrt(v + eps) * g + b


def _gelu(x):
    return x * 0.5 * (1.0 + jax.lax.erf(x * 0.7071067811865476))


def _edge_body(feats_ref, verts_ref, nbr_ref,
               w1a_ref, w1b_ref, w1c_ref, b1_ref, g1_ref, bt1_ref,
               w2_ref, b2_ref,
               wsa_ref, wsb_ref, wsc_ref, bs_ref, g2_ref, bt2_ref,
               ow1_ref, ob1_ref, og1_ref, obt1_ref,
               ow2_ref, ob2_ref, og2_ref, obt2_ref,
               out_ref):
    E = Q_TILE * K
    self_f = feats_ref[...]
    qv = verts_ref[...]
    nbr = nbr_ref[...]
    nbrF = nbr[:, :C_IN]
    nbrV = nbr[:, C_IN:C_IN + 3]

    selfe = jnp.broadcast_to(self_f[:, None, :], (Q_TILE, K, C_IN)).reshape(E, C_IN)
    rel = (nbrV.reshape(Q_TILE, K, 3) - qv[:, None, :]).reshape(E, 3)

    h = (jnp.dot(selfe, w1a_ref[...], preferred_element_type=jnp.float32)
         + jnp.dot(nbrF, w1b_ref[...], preferred_element_type=jnp.float32)
         + jnp.dot(rel, w1c_ref[...], preferred_element_type=jnp.float32)
         + b1_ref[...])
    h = _gelu(_ln(h, g1_ref[...], bt1_ref[...]))
    h2 = jnp.dot(h, w2_ref[...], preferred_element_type=jnp.float32) + b2_ref[...]
    sc = (jnp.dot(selfe, wsa_ref[...], preferred_element_type=jnp.float32)
          + jnp.dot(nbrF, wsb_ref[...], preferred_element_type=jnp.float32)
          + jnp.dot(rel, wsc_ref[...], preferred_element_type=jnp.float32)
          + bs_ref[...])
    e = _gelu(_ln(h2 + sc, g2_ref[...], bt2_ref[...]))

    red = jnp.mean(e.reshape(Q_TILE, K, C_OUT), axis=1)

    oh = jnp.dot(red, ow1_ref[...], preferred_element_type=jnp.float32) + ob1_ref[...]
    oh = _gelu(_ln(oh, og1_ref[...], obt1_ref[...]))
    oh2 = jnp.dot(oh, ow2_ref[...], preferred_element_type=jnp.float32) + ob2_ref[...]
    out_ref[...] = _gelu(_ln(oh2 + red, og2_ref[...], obt2_ref[...]))


def _rep(shape):
    return pl.BlockSpec(shape, lambda i: (0,) * len(shape))


def _edge_pallas(feats, verts, nbr_packed, weights):
    E = Q_TILE * K
    in_specs = [
        pl.BlockSpec((Q_TILE, C_IN), lambda i: (i, 0)),
        pl.BlockSpec((Q_TILE, 3), lambda i: (i, 0)),
        pl.BlockSpec((E, PACK), lambda i: (i, 0)),
    ] + [_rep(w.shape) for w in weights]
    return pl.pallas_call(
        _edge_body,
        grid=(GRID,),
        in_specs=in_specs,
        out_specs=pl.BlockSpec((Q_TILE, C_OUT), lambda i: (i, 0)),
        out_shape=jax.ShapeDtypeStruct((N, C_OUT), jnp.float32),
    )(feats, verts, nbr_packed, *weights)


def _sc_gather(table, idx):
    B = idx.shape[0]
    nch = B // GCH
    base = nch // SC_NW
    extra = nch - base * SC_NW
    mesh = plsc.VectorSubcoreMesh(core_axis_name="c", subcore_axis_name="s")

    @functools.partial(
        pl.kernel, mesh=mesh,
        out_type=jax.ShapeDtypeStruct((B, PACK), jnp.float32),
        scratch_types=[
            pltpu.VMEM((GCH,), jnp.int32),
            pltpu.VMEM((GCH, PACK), jnp.float32),
            pltpu.SemaphoreType.DMA,
        ],
    )
    def k(table_hbm, idx_hbm, out_hbm, idx_v, rows_v, sem):
        wid = jax.lax.axis_index("s") * SC_NC + jax.lax.axis_index("c")
        trips = base + jnp.where(wid < extra, 1, 0)

        def body(i, carry):
            off = (wid + i * SC_NW) * GCH
            pltpu.sync_copy(idx_hbm.at[pl.ds(off, GCH)], idx_v)
            pltpu.async_copy(table_hbm.at[idx_v], rows_v, sem).wait()
            pltpu.sync_copy(rows_v, out_hbm.at[pl.ds(off, GCH)])
            return carry

        jax.lax.fori_loop(0, trips, body, 0)

    return k(table, idx)


def _knn_idx(in_v, k, chunk=2500, grp=1024, sub=10):
    vn = jnp.sum(in_v * in_v, axis=1)
    npad = grp * sub
    qs = in_v.reshape(N // chunk, chunk, 3)

    sup = 4

    def body(q):
        s = 2.0 * jnp.dot(q, in_v.T, precision=jax.lax.Precision.HIGHEST) - vn[None, :]
        sp = jnp.pad(s, ((0, 0), (0, npad - N)), constant_values=-jnp.inf)
        sg = sp.reshape(chunk, grp, sub)
        m = jnp.max(sg, axis=-1)
        m2 = jnp.max(m.reshape(chunk, grp // sup, sup), axis=-1)
        _, s_idx = jax.lax.top_k(m2, k)
        mcand = jnp.take_along_axis(
            m.reshape(chunk, grp // sup, sup), s_idx[..., None], axis=1)
        _, msel = jax.lax.top_k(mcand.reshape(chunk, k * sup), k)
        gidx = (jnp.take_along_axis(s_idx, msel // sup, axis=1) * sup
                + (msel % sup))
        cand = jnp.take_along_axis(sg, gidx[..., None], axis=1)
        _, sel = jax.lax.top_k(cand.reshape(chunk, k * sub), k)
        g = jnp.take_along_axis(gidx, sel // sub, axis=1)
        return g * sub + (sel % sub)

    idx = jax.lax.map(body, qs)
    return idx.reshape(N * k)


def kernel(vertices, features, e_w1, e_b1, e_g1, e_bt1, e_w2, e_b2, e_ws,
           e_bs, e_g2, e_bt2, o_w1, o_b1, o_g1, o_bt1, o_w2, o_b2, o_g2,
           o_bt2):
    B = vertices.shape[0]
    in_v = vertices.reshape(N, 3)
    feats = features.reshape(N, C_IN)

    idx = _knn_idx(in_v, K)
    table = jnp.concatenate(
        [feats, in_v, jnp.zeros((N, PACK - C_IN - 3), jnp.float32)], axis=1)
    nbr_packed = _sc_gather(table, idx)

    weights = (
        e_w1[:C_IN], e_w1[C_IN:2 * C_IN], e_w1[2 * C_IN:], e_b1, e_g1, e_bt1,
        e_w2, e_b2,
        e_ws[:C_IN], e_ws[C_IN:2 * C_IN], e_ws[2 * C_IN:], e_bs, e_g2, e_bt2,
        o_w1, o_b1, o_g1, o_bt1, o_w2, o_b2, o_g2, o_bt2,
    )
    out = _edge_pallas(feats, in_v, nbr_packed, weights)
    return out.reshape(B, N, C_OUT)


# speedup vs baseline: 5.5542x; 1.0111x over previous
import functools
import jax
import jax.numpy as jnp
from jax.experimental import pallas as pl
from jax.experimental.pallas import tpu as pltpu
from jax.experimental.pallas import tpu_sc as plsc

N = 10000
C_IN = 32
C_OUT = 32
HID = 64
K = 64
Q_TILE = 40
GRID = N // Q_TILE

SC_NC = 2
SC_NS = 16
SC_NW = SC_NC * SC_NS
GCH = 128
PACK = 128


def _ln(x, g, b, eps=1e-5):
    m = jnp.mean(x, axis=-1, keepdims=True)
    v = jnp.mean((x - m) * (x - m), axis=-1, keepdims=True)
    return (x - m) * jax.lax.rsqrt(v + eps) * g + b


def _gelu(x):
    return x * 0.5 * (1.0 + jax.lax.erf(x * 0.7071067811865476))


def _edge_body(feats_ref, verts_ref, nbr_ref,
               w1a_ref, w1b_ref, w1c_ref, b1_ref, g1_ref, bt1_ref,
               w2_ref, b2_ref,
               wsa_ref, wsb_ref, wsc_ref, bs_ref, g2_ref, bt2_ref,
               ow1_ref, ob1_ref, og1_ref, obt1_ref,
               ow2_ref, ob2_ref, og2_ref, obt2_ref,
               out_ref):
    E = Q_TILE * K
    self_f = feats_ref[...]
    qv = verts_ref[...]
    nbr = nbr_ref[...]
    nbrF = nbr[:, :C_IN]
    nbrV = nbr[:, C_IN:C_IN + 3]

    selfe = jnp.broadcast_to(self_f[:, None, :], (Q_TILE, K, C_IN)).reshape(E, C_IN)
    rel = (nbrV.reshape(Q_TILE, K, 3) - qv[:, None, :]).reshape(E, 3)

    h = (jnp.dot(selfe, w1a_ref[...], preferred_element_type=jnp.float32)
         + jnp.dot(nbrF, w1b_ref[...], preferred_element_type=jnp.float32)
         + jnp.dot(rel, w1c_ref[...], preferred_element_type=jnp.float32)
         + b1_ref[...])
    h = _gelu(_ln(h, g1_ref[...], bt1_ref[...]))
    h2 = jnp.dot(h, w2_ref[...], preferred_element_type=jnp.float32) + b2_ref[...]
    sc = (jnp.dot(selfe, wsa_ref[...], preferred_element_type=jnp.float32)
          + jnp.dot(nbrF, wsb_ref[...], preferred_element_type=jnp.float32)
          + jnp.dot(rel, wsc_ref[...], preferred_element_type=jnp.float32)
          + bs_ref[...])
    e = _gelu(_ln(h2 + sc, g2_ref[...], bt2_ref[...]))

    red = jnp.mean(e.reshape(Q_TILE, K, C_OUT), axis=1)

    oh = jnp.dot(red, ow1_ref[...], preferred_element_type=jnp.float32) + ob1_ref[...]
    oh = _gelu(_ln(oh, og1_ref[...], obt1_ref[...]))
    oh2 = jnp.dot(oh, ow2_ref[...], preferred_element_type=jnp.float32) + ob2_ref[...]
    out_ref[...] = _gelu(_ln(oh2 + red, og2_ref[...], obt2_ref[...]))


def _rep(shape):
    return pl.BlockSpec(shape, lambda i: (0,) * len(shape))


def _edge_pallas(feats, verts, nbr_packed, weights):
    E = Q_TILE * K
    in_specs = [
        pl.BlockSpec((Q_TILE, C_IN), lambda i: (i, 0)),
        pl.BlockSpec((Q_TILE, 3), lambda i: (i, 0)),
        pl.BlockSpec((E, PACK), lambda i: (i, 0)),
    ] + [_rep(w.shape) for w in weights]
    return pl.pallas_call(
        _edge_body,
        grid=(GRID,),
        in_specs=in_specs,
        out_specs=pl.BlockSpec((Q_TILE, C_OUT), lambda i: (i, 0)),
        out_shape=jax.ShapeDtypeStruct((N, C_OUT), jnp.float32),
    )(feats, verts, nbr_packed, *weights)


def _sc_gather(table, idx):
    B = idx.shape[0]
    nch = B // GCH
    base = nch // SC_NW
    extra = nch - base * SC_NW
    mesh = plsc.VectorSubcoreMesh(core_axis_name="c", subcore_axis_name="s")

    @functools.partial(
        pl.kernel, mesh=mesh,
        out_type=jax.ShapeDtypeStruct((B, PACK), jnp.float32),
        scratch_types=[
            pltpu.VMEM((GCH,), jnp.int32),
            pltpu.VMEM((GCH, PACK), jnp.float32),
            pltpu.SemaphoreType.DMA,
        ],
    )
    def k(table_hbm, idx_hbm, out_hbm, idx_v, rows_v, sem):
        wid = jax.lax.axis_index("s") * SC_NC + jax.lax.axis_index("c")
        trips = base + jnp.where(wid < extra, 1, 0)

        def body(i, carry):
            off = (wid + i * SC_NW) * GCH
            pltpu.sync_copy(idx_hbm.at[pl.ds(off, GCH)], idx_v)
            pltpu.async_copy(table_hbm.at[idx_v], rows_v, sem).wait()
            pltpu.sync_copy(rows_v, out_hbm.at[pl.ds(off, GCH)])
            return carry

        jax.lax.fori_loop(0, trips, body, 0)

    return k(table, idx)


def _knn_idx(in_v, k, chunk=5000, grp=1024, sub=10):
    vn = jnp.sum(in_v * in_v, axis=1)
    npad = grp * sub
    qs = in_v.reshape(N // chunk, chunk, 3)

    sup = 4

    def body(q):
        s = 2.0 * jnp.dot(q, in_v.T, precision=jax.lax.Precision.HIGHEST) - vn[None, :]
        sp = jnp.pad(s, ((0, 0), (0, npad - N)), constant_values=-jnp.inf)
        sg = sp.reshape(chunk, grp, sub)
        m = jnp.max(sg, axis=-1)
        m2 = jnp.max(m.reshape(chunk, grp // sup, sup), axis=-1)
        _, s_idx = jax.lax.top_k(m2, k)
        mcand = jnp.take_along_axis(
            m.reshape(chunk, grp // sup, sup), s_idx[..., None], axis=1)
        _, msel = jax.lax.top_k(mcand.reshape(chunk, k * sup), k)
        gidx = (jnp.take_along_axis(s_idx, msel // sup, axis=1) * sup
                + (msel % sup))
        cand = jnp.take_along_axis(sg, gidx[..., None], axis=1)
        _, sel = jax.lax.top_k(cand.reshape(chunk, k * sub), k)
        g = jnp.take_along_axis(gidx, sel // sub, axis=1)
        return g * sub + (sel % sub)

    idx = jax.lax.map(body, qs)
    return idx.reshape(N * k)


def kernel(vertices, features, e_w1, e_b1, e_g1, e_bt1, e_w2, e_b2, e_ws,
           e_bs, e_g2, e_bt2, o_w1, o_b1, o_g1, o_bt1, o_w2, o_b2, o_g2,
           o_bt2):
    B = vertices.shape[0]
    in_v = vertices.reshape(N, 3)
    feats = features.reshape(N, C_IN)

    idx = _knn_idx(in_v, K)
    table = jnp.concatenate(
        [feats, in_v, jnp.zeros((N, PACK - C_IN - 3), jnp.float32)], axis=1)
    nbr_packed = _sc_gather(table, idx)

    weights = (
        e_w1[:C_IN], e_w1[C_IN:2 * C_IN], e_w1[2 * C_IN:], e_b1, e_g1, e_bt1,
        e_w2, e_b2,
        e_ws[:C_IN], e_ws[C_IN:2 * C_IN], e_ws[2 * C_IN:], e_bs, e_g2, e_bt2,
        o_w1, o_b1, o_g1, o_bt1, o_w2, o_b2, o_g2, o_bt2,
    )
    out = _edge_pallas(feats, in_v, nbr_packed, weights)
    return out.reshape(B, N, C_OUT)
